# baseline (device time: 48862 ns/iter reference)
import jax
import jax.numpy as jnp
from jax import lax
from jax.experimental import pallas as pl
from jax.experimental.pallas import tpu as pltpu

N_RING = 4
CHUNK = 128


def kernel(x, dy):
    k_per, m = x.shape
    _, n = dy.shape

    def body(x_ref, dy_ref, out_ref, partial_ref, comm_ref, send_sems, recv_sems):
        xi = lax.axis_index("x")
        yi = lax.axis_index("y")
        zi = lax.axis_index("z")
        right = (zi + 1) % N_RING
        left = (zi - 1) % N_RING

        barrier_sem = pltpu.get_barrier_semaphore()
        for nbr in (left, right):
            pl.semaphore_signal(
                barrier_sem,
                inc=1,
                device_id=(xi, yi, nbr),
                device_id_type=pl.DeviceIdType.MESH,
            )
        pl.semaphore_wait(barrier_sem, 2)

        partial_ref[...] = lax.dot_general(
            x_ref[...],
            dy_ref[...],
            dimension_numbers=(((0,), (0,)), ((), ())),
            preferred_element_type=jnp.float32,
        )

        first = (zi - 1) % N_RING
        comm_ref[0, :, :] = partial_ref[pl.ds(first * CHUNK, CHUNK), :]

        for t in range(N_RING - 1):
            rdma = pltpu.make_async_remote_copy(
                src_ref=comm_ref.at[t],
                dst_ref=comm_ref.at[t + 1],
                send_sem=send_sems.at[t],
                recv_sem=recv_sems.at[t],
                device_id=(xi, yi, right),
                device_id_type=pl.DeviceIdType.MESH,
            )
            rdma.start()
            rdma.wait()

            c = (zi - 2 - t) % N_RING
            if t < N_RING - 2:
                comm_ref[t + 1, :, :] = (
                    comm_ref[t + 1, :, :] + partial_ref[pl.ds(c * CHUNK, CHUNK), :]
                )
            else:
                out_ref[...] = (
                    comm_ref[t + 1, :, :] + partial_ref[pl.ds(zi * CHUNK, CHUNK), :]
                )

    return pl.pallas_call(
        body,
        out_shape=jax.ShapeDtypeStruct((CHUNK, n), jnp.float32),
        in_specs=[
            pl.BlockSpec(memory_space=pltpu.VMEM),
            pl.BlockSpec(memory_space=pltpu.VMEM),
        ],
        out_specs=pl.BlockSpec(memory_space=pltpu.VMEM),
        scratch_shapes=[
            pltpu.VMEM((m, n), jnp.float32),
            pltpu.VMEM((N_RING, CHUNK, n), jnp.float32),
            pltpu.SemaphoreType.DMA((N_RING - 1,)),
            pltpu.SemaphoreType.DMA((N_RING - 1,)),
        ],
        compiler_params=pltpu.CompilerParams(collective_id=0),
    )(x, dy)


# device time: 47482 ns/iter; 1.0291x vs baseline; 1.0291x over previous
import jax
import jax.numpy as jnp
from jax import lax
from jax.experimental import pallas as pl
from jax.experimental.pallas import tpu as pltpu

N_RING = 4
CHUNK = 128


def kernel(x, dy):
    k_per, m = x.shape
    _, n = dy.shape
    half = n // 2

    def body(
        x_ref,
        dy_ref,
        out_ref,
        partial_ref,
        cw_ref,
        ccw_ref,
        cw_ssem,
        cw_rsem,
        ccw_ssem,
        ccw_rsem,
    ):
        xi = lax.axis_index("x")
        yi = lax.axis_index("y")
        zi = lax.axis_index("z")
        right = (zi + 1) % N_RING
        left = (zi - 1) % N_RING

        barrier_sem = pltpu.get_barrier_semaphore()
        for nbr in (left, right):
            pl.semaphore_signal(
                barrier_sem,
                inc=1,
                device_id=(xi, yi, nbr),
                device_id_type=pl.DeviceIdType.MESH,
            )
        pl.semaphore_wait(barrier_sem, 2)

        def compute_chunk(c):
            partial_ref[pl.ds(c * CHUNK, CHUNK), :] = lax.dot_general(
                x_ref[:, pl.ds(c * CHUNK, CHUNK)],
                dy_ref[...],
                dimension_numbers=(((0,), (0,)), ((), ())),
                preferred_element_type=jnp.float32,
            )

        compute_chunk(left)
        compute_chunk(right)
        cw_ref[0, :, :] = partial_ref[pl.ds(left * CHUNK, CHUNK), :half]
        ccw_ref[0, :, :] = partial_ref[pl.ds(right * CHUNK, CHUNK), half:]

        for t in range(N_RING - 1):
            cw_rdma = pltpu.make_async_remote_copy(
                src_ref=cw_ref.at[t],
                dst_ref=cw_ref.at[t + 1],
                send_sem=cw_ssem.at[t],
                recv_sem=cw_rsem.at[t],
                device_id=(xi, yi, right),
                device_id_type=pl.DeviceIdType.MESH,
            )
            ccw_rdma = pltpu.make_async_remote_copy(
                src_ref=ccw_ref.at[t],
                dst_ref=ccw_ref.at[t + 1],
                send_sem=ccw_ssem.at[t],
                recv_sem=ccw_rsem.at[t],
                device_id=(xi, yi, left),
                device_id_type=pl.DeviceIdType.MESH,
            )
            cw_rdma.start()
            ccw_rdma.start()

            if t == 0:
                compute_chunk((zi + 2) % N_RING)
                compute_chunk(zi)

            cw_rdma.wait()
            ccw_rdma.wait()

            c_cw = (zi - 2 - t) % N_RING
            c_ccw = (zi + 2 + t) % N_RING
            if t < N_RING - 2:
                cw_ref[t + 1, :, :] = (
                    cw_ref[t + 1, :, :] + partial_ref[pl.ds(c_cw * CHUNK, CHUNK), :half]
                )
                ccw_ref[t + 1, :, :] = (
                    ccw_ref[t + 1, :, :]
                    + partial_ref[pl.ds(c_ccw * CHUNK, CHUNK), half:]
                )
            else:
                out_ref[:, :half] = (
                    cw_ref[t + 1, :, :] + partial_ref[pl.ds(zi * CHUNK, CHUNK), :half]
                )
                out_ref[:, half:] = (
                    ccw_ref[t + 1, :, :] + partial_ref[pl.ds(zi * CHUNK, CHUNK), half:]
                )

    return pl.pallas_call(
        body,
        out_shape=jax.ShapeDtypeStruct((CHUNK, n), jnp.float32),
        in_specs=[
            pl.BlockSpec(memory_space=pltpu.VMEM),
            pl.BlockSpec(memory_space=pltpu.VMEM),
        ],
        out_specs=pl.BlockSpec(memory_space=pltpu.VMEM),
        scratch_shapes=[
            pltpu.VMEM((m, n), jnp.float32),
            pltpu.VMEM((N_RING, CHUNK, half), jnp.float32),
            pltpu.VMEM((N_RING, CHUNK, half), jnp.float32),
            pltpu.SemaphoreType.DMA((N_RING - 1,)),
            pltpu.SemaphoreType.DMA((N_RING - 1,)),
            pltpu.SemaphoreType.DMA((N_RING - 1,)),
            pltpu.SemaphoreType.DMA((N_RING - 1,)),
        ],
        compiler_params=pltpu.CompilerParams(collective_id=0),
    )(x, dy)


# device time: 6084 ns/iter; 8.0312x vs baseline; 7.8044x over previous
import jax
import jax.numpy as jnp
from jax import lax
from jax.experimental import pallas as pl
from jax.experimental.pallas import tpu as pltpu

N_RING = 4
CHUNK = 128


def kernel(x, dy):
    k_per, m = x.shape
    _, n = dy.shape

    def body(
        x_ref,
        dy_ref,
        out_ref,
        partial_ref,
        rrecv,
        lrecv,
        rssem,
        rrsem,
        lssem,
        lrsem,
    ):
        xi = lax.axis_index("x")
        yi = lax.axis_index("y")
        zi = lax.axis_index("z")
        right = (zi + 1) % N_RING
        left = (zi - 1) % N_RING

        barrier_sem = pltpu.get_barrier_semaphore()
        for nbr in (left, right):
            pl.semaphore_signal(
                barrier_sem,
                inc=1,
                device_id=(xi, yi, nbr),
                device_id_type=pl.DeviceIdType.MESH,
            )
        pl.semaphore_wait(barrier_sem, 2)

        def compute_chunk(c):
            partial_ref[pl.ds(c * CHUNK, CHUNK), :] = lax.dot_general(
                x_ref[:, pl.ds(c * CHUNK, CHUNK)],
                dy_ref[...],
                dimension_numbers=(((0,), (0,)), ((), ())),
                preferred_element_type=jnp.float32,
            )

        def rsend_edge(c):
            return pltpu.make_async_remote_copy(
                src_ref=partial_ref.at[pl.ds(c * CHUNK, CHUNK), :],
                dst_ref=rrecv.at[c],
                send_sem=rssem.at[c],
                recv_sem=rrsem.at[c],
                device_id=(xi, yi, right),
                device_id_type=pl.DeviceIdType.MESH,
            )

        def lsend_edge(c):
            return pltpu.make_async_remote_copy(
                src_ref=partial_ref.at[pl.ds(c * CHUNK, CHUNK), :],
                dst_ref=lrecv.at[c],
                send_sem=lssem.at[c],
                recv_sem=lrsem.at[c],
                device_id=(xi, yi, left),
                device_id_type=pl.DeviceIdType.MESH,
            )

        def rfwd(c):
            return pltpu.make_async_remote_copy(
                src_ref=rrecv.at[c],
                dst_ref=rrecv.at[c],
                send_sem=rssem.at[c],
                recv_sem=rrsem.at[c],
                device_id=(xi, yi, right),
                device_id_type=pl.DeviceIdType.MESH,
            )

        def lfwd(c):
            return pltpu.make_async_remote_copy(
                src_ref=lrecv.at[c],
                dst_ref=lrecv.at[c],
                send_sem=lssem.at[c],
                recv_sem=lrsem.at[c],
                device_id=(xi, yi, left),
                device_id_type=pl.DeviceIdType.MESH,
            )

        @pl.when(zi <= 1)
        def _():
            for c in (3, 2, 1):
                compute_chunk(c)

                @pl.when(zi == 0)
                def _():
                    rsend_edge(c).start()

            compute_chunk(0)

        @pl.when(zi >= 2)
        def _():
            for c in (0, 1, 2):
                compute_chunk(c)

                @pl.when(zi == 3)
                def _():
                    lsend_edge(c).start()

            compute_chunk(3)

        for s in (1, 2, 3):
            for rc in (1, 2, 3):
                z_m = rc + s - 3
                if 1 <= z_m <= rc:

                    @pl.when(zi == z_m)
                    def _(rc=rc, z_m=z_m):
                        rfwd(rc).wait_recv()
                        if rc > z_m:
                            rrecv[rc, :, :] = (
                                rrecv[rc, :, :]
                                + partial_ref[pl.ds(rc * CHUNK, CHUNK), :]
                            )
                            rfwd(rc).start()

            for lc in (0, 1, 2):
                z_m = lc + 3 - s
                if lc <= z_m <= 2:

                    @pl.when(zi == z_m)
                    def _(lc=lc, z_m=z_m):
                        lfwd(lc).wait_recv()
                        if lc < z_m:
                            lrecv[lc, :, :] = (
                                lrecv[lc, :, :]
                                + partial_ref[pl.ds(lc * CHUNK, CHUNK), :]
                            )
                            lfwd(lc).start()

        for z_s in range(N_RING):

            @pl.when(zi == z_s)
            def _(z_s=z_s):
                val = partial_ref[pl.ds(z_s * CHUNK, CHUNK), :]
                if z_s >= 1:
                    val = val + rrecv[z_s, :, :]
                if z_s <= 2:
                    val = val + lrecv[z_s, :, :]
                out_ref[...] = val

        @pl.when(zi == 0)
        def _():
            for c in (3, 2, 1):
                rsend_edge(c).wait_send()

        @pl.when(zi == 3)
        def _():
            for c in (0, 1, 2):
                lsend_edge(c).wait_send()

        for z_s in (1, 2):

            @pl.when(zi == z_s)
            def _(z_s=z_s):
                for rc in range(z_s + 1, N_RING):
                    rfwd(rc).wait_send()
                for lc in range(0, z_s):
                    lfwd(lc).wait_send()

    return pl.pallas_call(
        body,
        out_shape=jax.ShapeDtypeStruct((CHUNK, n), jnp.float32),
        in_specs=[
            pl.BlockSpec(memory_space=pltpu.VMEM),
            pl.BlockSpec(memory_space=pltpu.VMEM),
        ],
        out_specs=pl.BlockSpec(memory_space=pltpu.VMEM),
        scratch_shapes=[
            pltpu.VMEM((m, n), jnp.float32),
            pltpu.VMEM((N_RING, CHUNK, n), jnp.float32),
            pltpu.VMEM((N_RING, CHUNK, n), jnp.float32),
            pltpu.SemaphoreType.DMA((N_RING,)),
            pltpu.SemaphoreType.DMA((N_RING,)),
            pltpu.SemaphoreType.DMA((N_RING,)),
            pltpu.SemaphoreType.DMA((N_RING,)),
        ],
        compiler_params=pltpu.CompilerParams(collective_id=0),
    )(x, dy)
